# Initial kernel scaffold; baseline (speedup 1.0000x reference)
import jax
import jax.numpy as jnp
from jax.experimental import pallas as pl


def kernel(x, edge_attr, edge_index, g, W, b):
    raise NotImplementedError("write your pallas kernel here")


# speedup vs baseline: 6.5000x; 6.5000x over previous
import functools

import jax
import jax.numpy as jnp
from jax import lax
from jax.experimental import pallas as pl
from jax.experimental.pallas import tpu as pltpu
from jax.experimental.pallas import tpu_sc as plsc

N_NODES = 10000
N_EDGES = 320000
D_FEAT = 128
D_EDGE = 16

NC, NS = 2, 16
NW = NC * NS
EPW = N_EDGES // NW
SCHUNK = 80
LCHUNK = 2000
NLOAD = EPW // LCHUNK
SPL = LCHUNK // SCHUNK
NSEG = 10240
RPT = NSEG // NS


def _sc_body(dst3d, edge, acc_out, cnt_out, idx_v, ebuf, ones_v, zbuf,
             acc_sh, cnt_sh):
    c = lax.axis_index("c")
    s = lax.axis_index("s")
    w = s * NC + c

    zero16 = jnp.zeros((16,), jnp.float32)
    one16 = jnp.ones((16,), jnp.float32)

    @pl.loop(0, RPT)
    def _(i):
        zbuf[i, :] = zero16

    @pl.loop(0, SCHUNK)
    def _(i):
        ones_v[i, :] = one16

    pltpu.sync_copy(zbuf, acc_sh.at[pl.ds(s * RPT, RPT)])
    pltpu.sync_copy(zbuf, cnt_sh.at[pl.ds(s * RPT, RPT)])
    pltpu.sync_copy(dst3d.at[w], idx_v)
    plsc.subcore_barrier()

    for l in range(NLOAD):
        pltpu.sync_copy(edge.at[pl.ds(w * EPW + l * LCHUNK, LCHUNK)], ebuf)

        @pl.loop(0, SPL)
        def _(j):
            idx_row = idx_v.at[l * SPL + j]
            pltpu.sync_copy(ebuf.at[pl.ds(j * SCHUNK, SCHUNK)],
                            acc_sh.at[idx_row], add=True)
            pltpu.sync_copy(ones_v, cnt_sh.at[idx_row], add=True)

    plsc.subcore_barrier()
    pltpu.sync_copy(acc_sh.at[pl.ds(s * RPT, RPT)],
                    acc_out.at[c, pl.ds(s * RPT, RPT)])
    pltpu.sync_copy(cnt_sh.at[pl.ds(s * RPT, RPT)],
                    cnt_out.at[c, pl.ds(s * RPT, RPT)])


_sc_scatter = functools.partial(
    pl.kernel,
    out_type=(
        jax.ShapeDtypeStruct((NC, NSEG, D_EDGE), jnp.float32),
        jax.ShapeDtypeStruct((NC, NSEG, D_EDGE), jnp.float32),
    ),
    mesh=plsc.VectorSubcoreMesh(core_axis_name="c", subcore_axis_name="s",
                                num_cores=NC, num_subcores=NS),
    compiler_params=pltpu.CompilerParams(use_tc_tiling_on_sc=False),
    scratch_types=(
        pltpu.VMEM((EPW // SCHUNK, SCHUNK), jnp.int32),
        pltpu.VMEM((LCHUNK, D_EDGE), jnp.float32),
        pltpu.VMEM((SCHUNK, D_EDGE), jnp.float32),
        pltpu.VMEM((RPT, D_EDGE), jnp.float32),
        pltpu.VMEM_SHARED((NSEG, D_EDGE), jnp.float32),
        pltpu.VMEM_SHARED((NSEG, D_EDGE), jnp.float32),
    ),
)(_sc_body)


ROWS = 2000


def _tc_body(x_ref, acc_ref, cnt_ref, g_ref, W_ref, b_ref, o_ref):
    psum = acc_ref[0] + acc_ref[1]
    cnt = cnt_ref[0, :, 0:1] + cnt_ref[1, :, 0:1]
    agg = psum / jnp.maximum(cnt, 1.0)
    const = (jnp.dot(g_ref[...], W_ref[D_EDGE + D_FEAT:, :],
                     preferred_element_type=jnp.float32) + b_ref[...])
    o_ref[...] = (
        jnp.dot(agg, W_ref[0:D_EDGE, :], preferred_element_type=jnp.float32)
        + jnp.dot(x_ref[...], W_ref[D_EDGE:D_EDGE + D_FEAT, :],
                  preferred_element_type=jnp.float32)
        + const)


_tc_finish = pl.pallas_call(
    _tc_body,
    grid=(N_NODES // ROWS,),
    in_specs=[
        pl.BlockSpec((ROWS, D_FEAT), lambda i: (i, 0)),
        pl.BlockSpec((NC, ROWS, D_EDGE), lambda i: (0, i, 0)),
        pl.BlockSpec((NC, ROWS, D_EDGE), lambda i: (0, i, 0)),
        pl.BlockSpec((1, D_FEAT), lambda i: (0, 0)),
        pl.BlockSpec((D_EDGE + D_FEAT + D_FEAT, D_FEAT), lambda i: (0, 0)),
        pl.BlockSpec((1, D_FEAT), lambda i: (0, 0)),
    ],
    out_specs=pl.BlockSpec((ROWS, D_FEAT), lambda i: (i, 0)),
    out_shape=jax.ShapeDtypeStruct((N_NODES, D_FEAT), jnp.float32),
)


def kernel(x, edge_attr, edge_index, g, W, b):
    dst3d = edge_index[1].reshape(NW, EPW // SCHUNK, SCHUNK)
    acc, cnt = _sc_scatter(dst3d, edge_attr)
    return _tc_finish(x, acc, cnt, g.reshape(1, D_FEAT), W,
                      b.reshape(1, D_FEAT))
